# baseline (device time: 70653 ns/iter reference)
import jax
import jax.numpy as jnp
from jax import lax
from jax.experimental import pallas as pl
from jax.experimental.pallas import tpu as pltpu

N_DEV = 4
S = 8
W_TILES = 8


def kernel(x, w_mat):
    k_full, k_per = x.shape
    _, n = w_mat.shape
    m_per = k_full // N_DEV
    mc = m_per // S
    wt = k_full // W_TILES

    def body(x_ref, w_ref, out_ref, xs_ref, fstage_ref, xg_ref,
             wstage_ref, wb_ref, amax_ref,
             send_sems, recv_sems, a_send_sems, a_recv_sems,
             f_sems, w_sems):
        my = lax.axis_index("i")

        barrier_sem = pltpu.get_barrier_semaphore()
        for d in range(1, N_DEV):
            pl.semaphore_signal(
                barrier_sem, inc=1,
                device_id=((my + d) % N_DEV,),
                device_id_type=pl.DeviceIdType.MESH,
            )
        pl.semaphore_wait(barrier_sem, N_DEV - 1)

        jobs = []
        for c in range(S):
            for d in range(1, N_DEV):
                jobs.append((d, c))
            if c == 0:
                for cc in range(S):
                    jobs.append((0, cc))

        def job_dma(j, slot):
            d, c = jobs[j]
            dst = (my + d) % N_DEV
            return pltpu.make_async_copy(
                x_ref.at[pl.ds(dst * m_per + c * mc, mc)],
                fstage_ref.at[slot],
                f_sems.at[slot],
            )

        rdmas = {}
        job_dma(0, 0).start()
        job_dma(1, 1).start()
        for j in range(len(jobs)):
            d, c = jobs[j]
            job_dma(j, j % 2).wait()
            xs_ref[d, pl.ds(c * mc, mc)] = fstage_ref[j % 2].astype(
                jnp.bfloat16)
            if j + 2 < len(jobs):
                job_dma(j + 2, j % 2).start()
            if d > 0:
                dst = (my + d) % N_DEV
                rdma = pltpu.make_async_remote_copy(
                    src_ref=xs_ref.at[d, pl.ds(c * mc, mc)],
                    dst_ref=xg_ref.at[d - 1, pl.ds(c * mc, mc)],
                    send_sem=send_sems.at[d - 1, c],
                    recv_sem=recv_sems.at[d - 1, c],
                    device_id=(dst,),
                    device_id_type=pl.DeviceIdType.MESH,
                )
                rdma.start()
                rdmas[(d, c)] = rdma

        def w_dma(t, slot):
            return pltpu.make_async_copy(
                w_ref.at[pl.ds(t * wt, wt)],
                wstage_ref.at[slot],
                w_sems.at[slot],
            )

        w_dma(0, 0).start()
        w_dma(1, 1).start()
        for t in range(W_TILES):
            w_dma(t, t % 2).wait()
            wb_ref[pl.ds(t * wt, wt)] = wstage_ref[t % 2].astype(
                jnp.bfloat16)
            if t + 2 < W_TILES:
                w_dma(t + 2, t % 2).start()

        m_run = jnp.float32(0.0)
        for c in range(S):
            rows = pl.ds(c * mc, mc)
            acc = jnp.dot(
                xs_ref[0, rows],
                wb_ref[pl.ds((my % N_DEV) * k_per, k_per)],
                preferred_element_type=jnp.float32,
            )
            for d in range(1, N_DEV):
                rdmas[(d, c)].wait_recv()
                src = (my - d) % N_DEV
                acc = acc + jnp.dot(
                    xg_ref[d - 1, rows],
                    wb_ref[pl.ds(src * k_per, k_per)],
                    preferred_element_type=jnp.float32,
                )
            acc = jnp.maximum(acc, 0.0)
            m_run = jnp.maximum(m_run, jnp.max(acc))
            out_ref[rows] = acc

        for r in rdmas.values():
            r.wait_send()

        amax_ref[0] = jnp.full((1, 128), m_run, jnp.float32)
        a_rdmas = []
        for d in range(1, N_DEV):
            dst = (my + d) % N_DEV
            a_rdma = pltpu.make_async_remote_copy(
                src_ref=amax_ref.at[0],
                dst_ref=amax_ref.at[d],
                send_sem=a_send_sems.at[d],
                recv_sem=a_recv_sems.at[d],
                device_id=(dst,),
                device_id_type=pl.DeviceIdType.MESH,
            )
            a_rdma.start()
            a_rdmas.append(a_rdma)
        for a_rdma in a_rdmas:
            a_rdma.wait_recv()
        for a_rdma in a_rdmas:
            a_rdma.wait_send()

        scale = jnp.max(amax_ref[...]) / 448.0
        q = (out_ref[...] * (1.0 / scale)).astype(jnp.float8_e4m3fn)
        out_ref[...] = q.astype(jnp.float32) * scale

    return pl.pallas_call(
        body,
        out_shape=jax.ShapeDtypeStruct((m_per, n), jnp.float32),
        in_specs=[
            pl.BlockSpec(memory_space=pl.ANY),
            pl.BlockSpec(memory_space=pl.ANY),
        ],
        out_specs=pl.BlockSpec(memory_space=pltpu.VMEM),
        scratch_shapes=[
            pltpu.VMEM((N_DEV, m_per, k_per), jnp.bfloat16),
            pltpu.VMEM((2, mc, k_per), jnp.float32),
            pltpu.VMEM((N_DEV - 1, m_per, k_per), jnp.bfloat16),
            pltpu.VMEM((2, wt, n), jnp.float32),
            pltpu.VMEM((k_full, n), jnp.bfloat16),
            pltpu.VMEM((N_DEV, 1, 128), jnp.float32),
            pltpu.SemaphoreType.DMA((N_DEV - 1, S)),
            pltpu.SemaphoreType.DMA((N_DEV - 1, S)),
            pltpu.SemaphoreType.DMA((N_DEV,)),
            pltpu.SemaphoreType.DMA((N_DEV,)),
            pltpu.SemaphoreType.DMA((2,)),
            pltpu.SemaphoreType.DMA((2,)),
        ],
        compiler_params=pltpu.CompilerParams(
            collective_id=0,
            vmem_limit_bytes=56 * 1024 * 1024,
        ),
    )(x, w_mat)


# device time: 67656 ns/iter; 1.0443x vs baseline; 1.0443x over previous
import jax
import jax.numpy as jnp
from jax import lax
from jax.experimental import pallas as pl
from jax.experimental.pallas import tpu as pltpu

N_DEV = 4
S = 4
W_TILES = 8


def kernel(x, w_mat):
    k_full, k_per = x.shape
    _, n = w_mat.shape
    m_per = k_full // N_DEV
    mc = m_per // S
    wt = k_full // W_TILES

    def body(x_ref, w_ref, out_ref, xs_ref, fstage_ref, xg_ref,
             wstage_ref, wb_ref, amax_ref,
             send_sems, recv_sems, a_send_sems, a_recv_sems,
             f_sems, w_sems):
        my = lax.axis_index("i")

        barrier_sem = pltpu.get_barrier_semaphore()
        for d in range(1, N_DEV):
            pl.semaphore_signal(
                barrier_sem, inc=1,
                device_id=((my + d) % N_DEV,),
                device_id_type=pl.DeviceIdType.MESH,
            )
        pl.semaphore_wait(barrier_sem, N_DEV - 1)

        jobs = []
        for c in range(S):
            for d in range(1, N_DEV):
                jobs.append((d, c))
            if c == 0:
                for cc in range(S):
                    jobs.append((0, cc))

        def job_dma(j, slot):
            d, c = jobs[j]
            dst = (my + d) % N_DEV
            return pltpu.make_async_copy(
                x_ref.at[pl.ds(dst * m_per + c * mc, mc)],
                fstage_ref.at[slot],
                f_sems.at[slot],
            )

        rdmas = {}
        job_dma(0, 0).start()
        job_dma(1, 1).start()
        for j in range(len(jobs)):
            d, c = jobs[j]
            job_dma(j, j % 2).wait()
            xs_ref[d, pl.ds(c * mc, mc)] = fstage_ref[j % 2].astype(
                jnp.bfloat16)
            if j + 2 < len(jobs):
                job_dma(j + 2, j % 2).start()
            if d > 0:
                dst = (my + d) % N_DEV
                rdma = pltpu.make_async_remote_copy(
                    src_ref=xs_ref.at[d, pl.ds(c * mc, mc)],
                    dst_ref=xg_ref.at[d - 1, pl.ds(c * mc, mc)],
                    send_sem=send_sems.at[d - 1, c],
                    recv_sem=recv_sems.at[d - 1, c],
                    device_id=(dst,),
                    device_id_type=pl.DeviceIdType.MESH,
                )
                rdma.start()
                rdmas[(d, c)] = rdma

        def w_dma(t, slot):
            return pltpu.make_async_copy(
                w_ref.at[pl.ds(t * wt, wt)],
                wstage_ref.at[slot],
                w_sems.at[slot],
            )

        w_dma(0, 0).start()
        w_dma(1, 1).start()
        for t in range(W_TILES):
            w_dma(t, t % 2).wait()
            wb_ref[pl.ds(t * wt, wt)] = wstage_ref[t % 2].astype(
                jnp.bfloat16)
            if t + 2 < W_TILES:
                w_dma(t + 2, t % 2).start()

        m_run = jnp.float32(0.0)
        for c in range(S):
            rows = pl.ds(c * mc, mc)
            acc = jnp.dot(
                xs_ref[0, rows],
                wb_ref[pl.ds((my % N_DEV) * k_per, k_per)],
                preferred_element_type=jnp.float32,
            )
            for d in range(1, N_DEV):
                rdmas[(d, c)].wait_recv()
                src = (my - d) % N_DEV
                acc = acc + jnp.dot(
                    xg_ref[d - 1, rows],
                    wb_ref[pl.ds(src * k_per, k_per)],
                    preferred_element_type=jnp.float32,
                )
            acc = jnp.maximum(acc, 0.0)
            m_run = jnp.maximum(m_run, jnp.max(acc))
            out_ref[rows] = acc

        for r in rdmas.values():
            r.wait_send()

        amax_ref[0] = jnp.full((1, 128), m_run, jnp.float32)
        a_rdmas = []
        for d in range(1, N_DEV):
            dst = (my + d) % N_DEV
            a_rdma = pltpu.make_async_remote_copy(
                src_ref=amax_ref.at[0],
                dst_ref=amax_ref.at[d],
                send_sem=a_send_sems.at[d],
                recv_sem=a_recv_sems.at[d],
                device_id=(dst,),
                device_id_type=pl.DeviceIdType.MESH,
            )
            a_rdma.start()
            a_rdmas.append(a_rdma)
        for a_rdma in a_rdmas:
            a_rdma.wait_recv()
        for a_rdma in a_rdmas:
            a_rdma.wait_send()

        scale = jnp.max(amax_ref[...]) / 448.0
        q = (out_ref[...] * (1.0 / scale)).astype(jnp.float8_e4m3fn)
        out_ref[...] = q.astype(jnp.float32) * scale

    return pl.pallas_call(
        body,
        out_shape=jax.ShapeDtypeStruct((m_per, n), jnp.float32),
        in_specs=[
            pl.BlockSpec(memory_space=pl.ANY),
            pl.BlockSpec(memory_space=pl.ANY),
        ],
        out_specs=pl.BlockSpec(memory_space=pltpu.VMEM),
        scratch_shapes=[
            pltpu.VMEM((N_DEV, m_per, k_per), jnp.bfloat16),
            pltpu.VMEM((2, mc, k_per), jnp.float32),
            pltpu.VMEM((N_DEV - 1, m_per, k_per), jnp.bfloat16),
            pltpu.VMEM((2, wt, n), jnp.float32),
            pltpu.VMEM((k_full, n), jnp.bfloat16),
            pltpu.VMEM((N_DEV, 1, 128), jnp.float32),
            pltpu.SemaphoreType.DMA((N_DEV - 1, S)),
            pltpu.SemaphoreType.DMA((N_DEV - 1, S)),
            pltpu.SemaphoreType.DMA((N_DEV,)),
            pltpu.SemaphoreType.DMA((N_DEV,)),
            pltpu.SemaphoreType.DMA((2,)),
            pltpu.SemaphoreType.DMA((2,)),
        ],
        compiler_params=pltpu.CompilerParams(
            collective_id=0,
            vmem_limit_bytes=56 * 1024 * 1024,
        ),
    )(x, w_mat)
